# baseline (speedup 1.0000x reference)
import functools

import jax
import jax.numpy as jnp
from jax import lax
from jax.experimental import pallas as pl
from jax.experimental.pallas import tpu as pltpu
from jax.experimental.pallas import tpu_sc as plsc

NC = 2
NS = 16
NW = NC * NS
LANES = 16
CHUNK = 128
GROUP = 4
N_ACC = 10240
DH = 64

_MESH = plsc.VectorSubcoreMesh(core_axis_name="c", subcore_axis_name="s")
_SC_PARAMS = pltpu.CompilerParams(use_tc_tiling_on_sc=False)


def _pipeline(k_chunks, gather, gather_wait, scatter):
    for b in range(GROUP):
        gather(b, b, 0)

    @pl.loop(0, k_chunks, step=2 * GROUP)
    def _(g):
        for b in range(GROUP):
            gather(g + GROUP + b, GROUP + b, 1)
        for b in range(GROUP):
            gather_wait(g + b, b, 0)
        sds = [scatter(g + b, b) for b in range(GROUP)]
        for dsc in sds:
            dsc.wait()
        for b in range(GROUP):
            gather(g + 2 * GROUP + b, b, 0)
        for b in range(GROUP):
            gather_wait(g + GROUP + b, GROUP + b, 1)
        sds = [scatter(g + GROUP + b, GROUP + b) for b in range(GROUP)]
        for dsc in sds:
            dsc.wait()

    for b in range(GROUP):
        gather_wait(k_chunks + b, b, 0)


def _row_agg_factory(n, k_chunks):
    rows_per_tile = N_ACC // NS
    assert rows_per_tile % 128 == 0
    k_alloc = k_chunks + GROUP

    @functools.partial(
        pl.kernel,
        out_type=jax.ShapeDtypeStruct((NC, N_ACC, DH), jnp.float32),
        mesh=_MESH,
        compiler_params=_SC_PARAMS,
        scratch_types=[
            pltpu.VMEM((k_alloc, CHUNK), jnp.int32),
            pltpu.VMEM((k_chunks, CHUNK), jnp.int32),
            pltpu.VMEM((2 * GROUP, CHUNK, DH), jnp.float32),
            pltpu.VMEM_SHARED((N_ACC, DH), jnp.float32),
            pltpu.SemaphoreType.DMA,
            pltpu.SemaphoreType.DMA,
            pltpu.SemaphoreType.DMA,
        ],
    )
    def agg(h_hbm, srcs_hbm, dsts_hbm, out_hbm, src_v, dst_v, rows,
            acc_sh, gsem0, gsem1, ssem):
        cid = lax.axis_index("c")
        sid = lax.axis_index("s")
        wid = sid * NC + cid
        gsems = [gsem0, gsem1]

        pltpu.sync_copy(srcs_hbm.at[wid], src_v)
        pltpu.sync_copy(dsts_hbm.at[wid], dst_v)

        def gather(j, b, h):
            pltpu.async_copy(h_hbm.at[src_v.at[j]], rows.at[b], gsems[h])

        def gather_wait(j, b, h):
            pltpu.make_async_copy(
                h_hbm.at[src_v.at[j]], rows.at[b], gsems[h]).wait()

        def scatter(j, b):
            return pltpu.async_copy(
                rows.at[b], acc_sh.at[dst_v.at[j]], ssem, add=True)

        zf = jnp.zeros((LANES,), jnp.float32)

        @pl.loop(0, CHUNK)
        def _(r):
            for c in range(DH // LANES):
                rows[0, r, pl.ds(c * LANES, LANES)] = zf

        base = sid * rows_per_tile
        for t in range(rows_per_tile // CHUNK):
            pltpu.sync_copy(rows.at[0],
                            acc_sh.at[pl.ds(base + t * CHUNK, CHUNK)])
        plsc.subcore_barrier()

        _pipeline(k_chunks, gather, gather_wait, scatter)

        plsc.subcore_barrier()
        pltpu.sync_copy(acc_sh.at[pl.ds(base, rows_per_tile)],
                        out_hbm.at[cid, pl.ds(base, rows_per_tile)])

    return agg


def _scalar_agg_factory(n, k_chunks, with_gather):
    per_tile = N_ACC // NS
    assert per_tile % LANES == 0
    k_alloc = k_chunks + GROUP

    scratch = [
        pltpu.VMEM((k_alloc, CHUNK), jnp.int32),
        pltpu.VMEM((k_chunks, CHUNK), jnp.int32),
        pltpu.VMEM((2 * GROUP, CHUNK), jnp.float32),
        pltpu.VMEM((per_tile,), jnp.float32),
        pltpu.VMEM_SHARED((N_ACC,), jnp.float32),
        pltpu.SemaphoreType.DMA,
        pltpu.SemaphoreType.DMA,
        pltpu.SemaphoreType.DMA,
    ]

    def body(vals_hbm, gidx_hbm, sidx_hbm, out_hbm, gidx_v, sidx_v, vals,
             zeros_v, acc_sh, gsem0, gsem1, ssem):
        cid = lax.axis_index("c")
        sid = lax.axis_index("s")
        wid = sid * NC + cid
        gsems = [gsem0, gsem1]

        if with_gather:
            pltpu.sync_copy(gidx_hbm.at[wid], gidx_v)
        pltpu.sync_copy(sidx_hbm.at[wid], sidx_v)

        def gather(j, b, h):
            pltpu.async_copy(vals_hbm.at[gidx_v.at[j]], vals.at[b], gsems[h])

        def gather_wait(j, b, h):
            pltpu.make_async_copy(
                vals_hbm.at[gidx_v.at[j]], vals.at[b], gsems[h]).wait()

        def scatter(j, b):
            return pltpu.async_copy(
                vals.at[b], acc_sh.at[sidx_v.at[j]], ssem, add=True)

        zf = jnp.zeros((LANES,), jnp.float32)

        @pl.loop(0, per_tile // LANES)
        def _(r):
            zeros_v[pl.ds(r * LANES, LANES)] = zf

        if not with_gather:
            one = jnp.ones((LANES,), jnp.float32)
            for b in range(2 * GROUP):
                for c in range(CHUNK // LANES):
                    vals[b, pl.ds(c * LANES, LANES)] = one

        base = sid * per_tile
        pltpu.sync_copy(zeros_v, acc_sh.at[pl.ds(base, per_tile)])
        plsc.subcore_barrier()

        if with_gather:
            _pipeline(k_chunks, gather, gather_wait, scatter)
        else:
            @pl.loop(0, k_chunks, step=2 * GROUP)
            def _(g):
                sds = [scatter(g + b, b) for b in range(2 * GROUP)]
                for dsc in sds:
                    dsc.wait()

        plsc.subcore_barrier()
        pltpu.sync_copy(acc_sh.at[pl.ds(base, per_tile)],
                        out_hbm.at[cid, pl.ds(base, per_tile)])

    return functools.partial(
        pl.kernel,
        out_type=jax.ShapeDtypeStruct((NC, N_ACC), jnp.float32),
        mesh=_MESH,
        compiler_params=_SC_PARAMS,
        scratch_types=scratch,
    )(body)


def _k1(degp3, x, w0, bn):
    n, d_in = x.shape
    d_h = w0.shape[1]
    grid = n // bn

    def body(deg_ref, x_ref, w_ref, ha_ref, hb_ref, dis_ref):
        deg = deg_ref[0, :, 0] + deg_ref[1, :, 0] + 1.0
        dis = lax.rsqrt(deg)
        h = jnp.dot(x_ref[...], w_ref[...], preferred_element_type=jnp.float32)
        hp = h * dis[:, None]
        ha_ref[...] = hp[:, :DH]
        hb_ref[...] = hp[:, DH:]
        dis_ref[...] = dis[:, None]

    return pl.pallas_call(
        body,
        grid=(grid,),
        in_specs=[
            pl.BlockSpec((NC, bn, 1), lambda i: (0, i, 0)),
            pl.BlockSpec((bn, d_in), lambda i: (i, 0)),
            pl.BlockSpec((d_in, d_h), lambda i: (0, 0)),
        ],
        out_specs=[
            pl.BlockSpec((bn, DH), lambda i: (i, 0)),
            pl.BlockSpec((bn, DH), lambda i: (i, 0)),
            pl.BlockSpec((bn, 1), lambda i: (i, 0)),
        ],
        out_shape=[
            jax.ShapeDtypeStruct((n, DH), jnp.float32),
            jax.ShapeDtypeStruct((n, DH), jnp.float32),
            jax.ShapeDtypeStruct((n, 1), jnp.float32),
        ],
    )(degp3, x, w0)


def _k2(apa, apb, hpa, hpb, dis, b, w, bn, split_out):
    n = hpa.shape[0]
    d = 2 * DH
    d_out = w.shape[1]
    grid = n // bn

    def body(apa_ref, apb_ref, hpa_ref, hpb_ref, dis_ref, b_ref, w_ref, *outs):
        agg = jnp.concatenate(
            [apa_ref[0] + apa_ref[1] + hpa_ref[...],
             apb_ref[0] + apb_ref[1] + hpb_ref[...]], axis=1)
        o = jnp.maximum(dis_ref[...] * agg + b_ref[...][None, :], 0.0)
        h = jnp.dot(o, w_ref[...], preferred_element_type=jnp.float32)
        h = h * dis_ref[...]
        if split_out:
            outs[0][...] = h[:, :DH]
            outs[1][...] = h[:, DH:]
        else:
            outs[0][...] = h

    if split_out:
        out_specs = [pl.BlockSpec((bn, DH), lambda i: (i, 0)),
                     pl.BlockSpec((bn, DH), lambda i: (i, 0))]
        out_shape = [jax.ShapeDtypeStruct((n, DH), jnp.float32),
                     jax.ShapeDtypeStruct((n, DH), jnp.float32)]
    else:
        out_specs = [pl.BlockSpec((bn, d_out), lambda i: (i, 0))]
        out_shape = [jax.ShapeDtypeStruct((n, d_out), jnp.float32)]

    return pl.pallas_call(
        body,
        grid=(grid,),
        in_specs=[
            pl.BlockSpec((NC, bn, DH), lambda i: (0, i, 0)),
            pl.BlockSpec((NC, bn, DH), lambda i: (0, i, 0)),
            pl.BlockSpec((bn, DH), lambda i: (i, 0)),
            pl.BlockSpec((bn, DH), lambda i: (i, 0)),
            pl.BlockSpec((bn, 1), lambda i: (i, 0)),
            pl.BlockSpec((d,), lambda i: (0,)),
            pl.BlockSpec((d, d_out), lambda i: (0, 0)),
        ],
        out_specs=out_specs,
        out_shape=out_shape,
    )(apa, apb, hpa, hpb, dis, b, w)


def _k4(a2p3, h2p, dis, b2, batch2, n, g):
    def body(a2_ref, h2_ref, dis_ref, b2_ref, bat_ref, out_ref):
        a2 = a2_ref[0, :n, 0] + a2_ref[1, :n, 0]
        out2 = dis_ref[:, 0] * (a2 + h2_ref[:, 0]) + b2_ref[0]
        gid = bat_ref[:, 0]
        oh = (gid[:, None] == lax.broadcasted_iota(jnp.int32, (1, g), 1)
              ).astype(jnp.float32)
        sums = lax.dot_general(oh, out2[:, None],
                               (((0,), (0,)), ((), ())),
                               preferred_element_type=jnp.float32)
        counts = jnp.sum(oh, axis=0)
        mean = sums[:, 0] / jnp.maximum(counts, 1.0)
        out_ref[...] = 1.0 / (1.0 + jnp.exp(-mean))

    return pl.pallas_call(
        body,
        out_shape=jax.ShapeDtypeStruct((g,), jnp.float32),
    )(a2p3, h2p, dis, b2, batch2)


def kernel(x, edge_index, batch, W0, b0, W1, b1, W2, b2):
    n, d_in = x.shape
    e = edge_index.shape[1]
    g = 64
    bn = 2000

    k_chunks = -(-e // (NW * CHUNK))
    k_chunks = -(-k_chunks // (2 * GROUP)) * (2 * GROUP)
    k_alloc = k_chunks + GROUP
    e_pad = NW * k_chunks * CHUNK
    src = edge_index[0]
    dst = edge_index[1]
    srcs = jnp.concatenate(
        [src, jnp.zeros((e_pad - e,), jnp.int32)]).reshape(NW, k_chunks, CHUNK)
    srcs = jnp.concatenate(
        [srcs, jnp.zeros((NW, GROUP, CHUNK), jnp.int32)], axis=1)
    dsts = jnp.concatenate(
        [dst, jnp.full((e_pad - e,), n, jnp.int32)]).reshape(NW, k_chunks, CHUNK)

    row_agg = _row_agg_factory(n, k_chunks)
    scalar_agg = _scalar_agg_factory(n, k_chunks, True)
    deg_count = _scalar_agg_factory(n, k_chunks, False)

    ones_pad = jnp.ones((N_ACC,), jnp.float32)
    degp = deg_count(ones_pad, srcs, dsts)

    h0a, h0b, dis = _k1(degp.reshape(NC, N_ACC, 1), x, W0, bn)
    a0a = row_agg(h0a, srcs, dsts)
    a0b = row_agg(h0b, srcs, dsts)
    h1a, h1b = _k2(a0a, a0b, h0a, h0b, dis, b0, W1, bn, True)
    a1a = row_agg(h1a, srcs, dsts)
    a1b = row_agg(h1b, srcs, dsts)
    h2p, = _k2(a1a, a1b, h1a, h1b, dis, b1, W2, bn, False)

    h2pad = jnp.concatenate([h2p[:, 0], jnp.zeros((N_ACC - n,), jnp.float32)])
    a2 = scalar_agg(h2pad, srcs, dsts)

    return _k4(a2.reshape(NC, N_ACC, 1), h2p, dis, b2,
               batch.reshape(n, 1), n, g)


# speedup vs baseline: 4.2416x; 4.2416x over previous
import functools

import jax
import jax.numpy as jnp
from jax import lax
from jax.experimental import pallas as pl
from jax.experimental.pallas import tpu as pltpu
from jax.experimental.pallas import tpu_sc as plsc

NC = 2
NS = 16
NW = NC * NS
LANES = 16
CHUNK = 128
GROUP = 4
N_ACC = 10240
DH = 64
QW = 32

_MESH = plsc.VectorSubcoreMesh(core_axis_name="c", subcore_axis_name="s")
_SC_PARAMS = pltpu.CompilerParams(use_tc_tiling_on_sc=False)


def _pipeline(k_chunks, gather, gather_wait, scatter):
    nb = 2 * GROUP

    @pl.loop(0, k_chunks, step=nb)
    def _(g):
        for b in range(nb):
            gather(g + b, b, 0)
        for b in range(nb):
            gather_wait(g + b, b, 0)
        sds = [scatter(g + b, b) for b in range(nb)]
        for dsc in sds:
            dsc.wait()


def _row_agg_factory(n, k_chunks):
    rows_per_tile = N_ACC // NS
    assert rows_per_tile % 128 == 0
    stage_rows = n // NS
    assert stage_rows * NS == n

    @functools.partial(
        pl.kernel,
        out_type=jax.ShapeDtypeStruct((NC, N_ACC, QW), jnp.float32),
        mesh=_MESH,
        compiler_params=_SC_PARAMS,
        scratch_types=[
            pltpu.VMEM((k_chunks, CHUNK), jnp.int32),
            pltpu.VMEM((k_chunks, CHUNK), jnp.int32),
            pltpu.VMEM((2 * GROUP, CHUNK, QW), jnp.float32),
            pltpu.VMEM_SHARED((N_ACC, QW), jnp.float32),
            pltpu.VMEM_SHARED((n, QW), jnp.float32),
            pltpu.SemaphoreType.DMA,
            pltpu.SemaphoreType.DMA,
            pltpu.SemaphoreType.DMA,
            pltpu.SemaphoreType.DMA,
        ],
    )
    def agg(h_hbm, srcs_hbm, dsts_hbm, out_hbm, src_v, dst_v, rows,
            acc_sh, h_sh, gsem0, gsem1, ssem, stsem):
        cid = lax.axis_index("c")
        sid = lax.axis_index("s")
        gsems = [gsem0, gsem1]

        sbase = sid * stage_rows
        stage_dma = pltpu.async_copy(
            h_hbm.at[pl.ds(sbase, stage_rows), pl.ds(cid * QW, QW)],
            h_sh.at[pl.ds(sbase, stage_rows)], stsem)

        pltpu.sync_copy(srcs_hbm.at[sid], src_v)
        pltpu.sync_copy(dsts_hbm.at[sid], dst_v)

        def gather(j, b, h):
            pltpu.async_copy(h_sh.at[src_v.at[j]], rows.at[b], gsems[h])

        def gather_wait(j, b, h):
            pltpu.make_async_copy(
                h_sh.at[src_v.at[j]], rows.at[b], gsems[h]).wait()

        def scatter(j, b):
            return pltpu.async_copy(
                rows.at[b], acc_sh.at[dst_v.at[j]], ssem, add=True)

        zf = jnp.zeros((LANES,), jnp.float32)

        @pl.loop(0, CHUNK)
        def _(r):
            for c in range(QW // LANES):
                rows[0, r, pl.ds(c * LANES, LANES)] = zf

        base = sid * rows_per_tile
        for t in range(rows_per_tile // CHUNK):
            pltpu.sync_copy(rows.at[0],
                            acc_sh.at[pl.ds(base + t * CHUNK, CHUNK)])
        stage_dma.wait()
        plsc.subcore_barrier()

        _pipeline(k_chunks, gather, gather_wait, scatter)

        plsc.subcore_barrier()
        pltpu.sync_copy(acc_sh.at[pl.ds(base, rows_per_tile)],
                        out_hbm.at[cid, pl.ds(base, rows_per_tile)])

    return agg


def _scalar_agg_factory(n, k_chunks, with_gather):
    per_tile = N_ACC // NS
    assert per_tile % LANES == 0
    k_alloc = k_chunks

    scratch = [
        pltpu.VMEM((k_alloc, CHUNK), jnp.int32),
        pltpu.VMEM((k_chunks, CHUNK), jnp.int32),
        pltpu.VMEM((2 * GROUP, CHUNK), jnp.float32),
        pltpu.VMEM((per_tile,), jnp.float32),
        pltpu.VMEM_SHARED((N_ACC,), jnp.float32),
        pltpu.SemaphoreType.DMA,
        pltpu.SemaphoreType.DMA,
        pltpu.SemaphoreType.DMA,
    ]

    def body(vals_hbm, gidx_hbm, sidx_hbm, out_hbm, gidx_v, sidx_v, vals,
             zeros_v, acc_sh, gsem0, gsem1, ssem):
        cid = lax.axis_index("c")
        sid = lax.axis_index("s")
        wid = sid * NC + cid
        gsems = [gsem0, gsem1]

        if with_gather:
            pltpu.sync_copy(gidx_hbm.at[wid], gidx_v)
        pltpu.sync_copy(sidx_hbm.at[wid], sidx_v)

        def gather(j, b, h):
            pltpu.async_copy(vals_hbm.at[gidx_v.at[j]], vals.at[b], gsems[h])

        def gather_wait(j, b, h):
            pltpu.make_async_copy(
                vals_hbm.at[gidx_v.at[j]], vals.at[b], gsems[h]).wait()

        def scatter(j, b):
            return pltpu.async_copy(
                vals.at[b], acc_sh.at[sidx_v.at[j]], ssem, add=True)

        zf = jnp.zeros((LANES,), jnp.float32)

        @pl.loop(0, per_tile // LANES)
        def _(r):
            zeros_v[pl.ds(r * LANES, LANES)] = zf

        if not with_gather:
            one = jnp.ones((LANES,), jnp.float32)
            for b in range(2 * GROUP):
                for c in range(CHUNK // LANES):
                    vals[b, pl.ds(c * LANES, LANES)] = one

        base = sid * per_tile
        pltpu.sync_copy(zeros_v, acc_sh.at[pl.ds(base, per_tile)])
        plsc.subcore_barrier()

        if with_gather:
            _pipeline(k_chunks, gather, gather_wait, scatter)
        else:
            @pl.loop(0, k_chunks, step=2 * GROUP)
            def _(g):
                sds = [scatter(g + b, b) for b in range(2 * GROUP)]
                for dsc in sds:
                    dsc.wait()

        plsc.subcore_barrier()
        pltpu.sync_copy(acc_sh.at[pl.ds(base, per_tile)],
                        out_hbm.at[cid, pl.ds(base, per_tile)])

    return functools.partial(
        pl.kernel,
        out_type=jax.ShapeDtypeStruct((NC, N_ACC), jnp.float32),
        mesh=_MESH,
        compiler_params=_SC_PARAMS,
        scratch_types=scratch,
    )(body)


def _k1(degp3, x, w0, bn):
    n, d_in = x.shape
    d_h = w0.shape[1]
    grid = n // bn

    def body(deg_ref, x_ref, w_ref, ha_ref, hb_ref, dis_ref):
        deg = deg_ref[0, :, 0] + deg_ref[1, :, 0] + 1.0
        dis = lax.rsqrt(deg)
        h = jnp.dot(x_ref[...], w_ref[...], preferred_element_type=jnp.float32)
        hp = h * dis[:, None]
        ha_ref[...] = hp[:, :DH]
        hb_ref[...] = hp[:, DH:]
        dis_ref[...] = dis[:, None]

    return pl.pallas_call(
        body,
        grid=(grid,),
        in_specs=[
            pl.BlockSpec((NC, bn, 1), lambda i: (0, i, 0)),
            pl.BlockSpec((bn, d_in), lambda i: (i, 0)),
            pl.BlockSpec((d_in, d_h), lambda i: (0, 0)),
        ],
        out_specs=[
            pl.BlockSpec((bn, DH), lambda i: (i, 0)),
            pl.BlockSpec((bn, DH), lambda i: (i, 0)),
            pl.BlockSpec((bn, 1), lambda i: (i, 0)),
        ],
        out_shape=[
            jax.ShapeDtypeStruct((n, DH), jnp.float32),
            jax.ShapeDtypeStruct((n, DH), jnp.float32),
            jax.ShapeDtypeStruct((n, 1), jnp.float32),
        ],
    )(degp3, x, w0)


def _k2(apa, apb, hpa, hpb, dis, b, w, bn, split_out):
    n = hpa.shape[0]
    d = 2 * DH
    d_out = w.shape[1]
    grid = n // bn

    def body(apa_ref, apb_ref, hpa_ref, hpb_ref, dis_ref, b_ref, w_ref, *outs):
        agg = jnp.concatenate(
            [apa_ref[0], apa_ref[1], apb_ref[0], apb_ref[1]], axis=1)
        agg = agg + jnp.concatenate([hpa_ref[...], hpb_ref[...]], axis=1)
        o = jnp.maximum(dis_ref[...] * agg + b_ref[...][None, :], 0.0)
        h = jnp.dot(o, w_ref[...], preferred_element_type=jnp.float32)
        h = h * dis_ref[...]
        if split_out:
            outs[0][...] = h[:, :DH]
            outs[1][...] = h[:, DH:]
        else:
            outs[0][...] = h

    if split_out:
        out_specs = [pl.BlockSpec((bn, DH), lambda i: (i, 0)),
                     pl.BlockSpec((bn, DH), lambda i: (i, 0))]
        out_shape = [jax.ShapeDtypeStruct((n, DH), jnp.float32),
                     jax.ShapeDtypeStruct((n, DH), jnp.float32)]
    else:
        out_specs = [pl.BlockSpec((bn, d_out), lambda i: (i, 0))]
        out_shape = [jax.ShapeDtypeStruct((n, d_out), jnp.float32)]

    return pl.pallas_call(
        body,
        grid=(grid,),
        in_specs=[
            pl.BlockSpec((NC, bn, QW), lambda i: (0, i, 0)),
            pl.BlockSpec((NC, bn, QW), lambda i: (0, i, 0)),
            pl.BlockSpec((bn, DH), lambda i: (i, 0)),
            pl.BlockSpec((bn, DH), lambda i: (i, 0)),
            pl.BlockSpec((bn, 1), lambda i: (i, 0)),
            pl.BlockSpec((d,), lambda i: (0,)),
            pl.BlockSpec((d, d_out), lambda i: (0, 0)),
        ],
        out_specs=out_specs,
        out_shape=out_shape,
    )(apa, apb, hpa, hpb, dis, b, w)


def _k4(a2p3, h2p, dis, b2, batch2, n, g):
    def body(a2_ref, h2_ref, dis_ref, b2_ref, bat_ref, out_ref):
        a2 = a2_ref[0, :n, 0] + a2_ref[1, :n, 0]
        out2 = dis_ref[:, 0] * (a2 + h2_ref[:, 0]) + b2_ref[0]
        gid = bat_ref[:, 0]
        oh = (gid[:, None] == lax.broadcasted_iota(jnp.int32, (1, g), 1)
              ).astype(jnp.float32)
        sums = lax.dot_general(oh, out2[:, None],
                               (((0,), (0,)), ((), ())),
                               preferred_element_type=jnp.float32)
        counts = jnp.sum(oh, axis=0)
        mean = sums[:, 0] / jnp.maximum(counts, 1.0)
        out_ref[...] = 1.0 / (1.0 + jnp.exp(-mean))

    return pl.pallas_call(
        body,
        out_shape=jax.ShapeDtypeStruct((g,), jnp.float32),
    )(a2p3, h2p, dis, b2, batch2)


def kernel(x, edge_index, batch, W0, b0, W1, b1, W2, b2):
    n, d_in = x.shape
    e = edge_index.shape[1]
    g = 64
    bn = 2000

    k_chunks = -(-e // (NW * CHUNK))
    k_chunks = -(-k_chunks // (2 * GROUP)) * (2 * GROUP)
    e_pad = NW * k_chunks * CHUNK
    k2 = 2 * k_chunks
    src = edge_index[0]
    dst = edge_index[1]
    src_flat = jnp.concatenate([src, jnp.zeros((e_pad - e,), jnp.int32)])
    dst_flat = jnp.concatenate([dst, jnp.full((e_pad - e,), n, jnp.int32)])
    srcs = src_flat.reshape(NW, k_chunks, CHUNK)
    dsts = dst_flat.reshape(NW, k_chunks, CHUNK)
    srcs2 = src_flat.reshape(NS, k2, CHUNK)
    dsts2 = dst_flat.reshape(NS, k2, CHUNK)

    row_agg = _row_agg_factory(n, k2)
    scalar_agg = _scalar_agg_factory(n, k_chunks, True)
    deg_count = _scalar_agg_factory(n, k_chunks, False)

    ones_pad = jnp.ones((N_ACC,), jnp.float32)
    degp = deg_count(ones_pad, srcs, dsts)

    h0a, h0b, dis = _k1(degp.reshape(NC, N_ACC, 1), x, W0, bn)
    a0a = row_agg(h0a, srcs2, dsts2)
    a0b = row_agg(h0b, srcs2, dsts2)
    h1a, h1b = _k2(a0a, a0b, h0a, h0b, dis, b0, W1, bn, True)
    a1a = row_agg(h1a, srcs2, dsts2)
    a1b = row_agg(h1b, srcs2, dsts2)
    h2p, = _k2(a1a, a1b, h1a, h1b, dis, b1, W2, bn, False)

    h2pad = jnp.concatenate([h2p[:, 0], jnp.zeros((N_ACC - n,), jnp.float32)])
    a2 = scalar_agg(h2pad, srcs, dsts)

    return _k4(a2.reshape(NC, N_ACC, 1), h2p, dis, b2,
               batch.reshape(n, 1), n, g)


# speedup vs baseline: 6.1685x; 1.4543x over previous
import functools

import jax
import jax.numpy as jnp
from jax import lax
from jax.experimental import pallas as pl
from jax.experimental.pallas import tpu as pltpu
from jax.experimental.pallas import tpu_sc as plsc

NC = 2
NS = 16
NW = NC * NS
LANES = 16
CHUNK = 128
GROUP = 4
N_ACC = 10240
DH = 64
QW = 32

_MESH = plsc.VectorSubcoreMesh(core_axis_name="c", subcore_axis_name="s")
_SC_PARAMS = pltpu.CompilerParams(use_tc_tiling_on_sc=False)


def _pipeline(k_chunks, gather, gather_wait, scatter):
    nb = 2 * GROUP

    @pl.loop(0, k_chunks, step=nb)
    def _(g):
        for b in range(nb):
            gather(g + b, b, 0)
        for b in range(nb):
            gather_wait(g + b, b, 0)
        sds = [scatter(g + b, b) for b in range(nb)]
        for dsc in sds:
            dsc.wait()


def _row_agg_factory(n, k_chunks):
    rows_per_tile = N_ACC // NS
    assert rows_per_tile % 128 == 0
    stage_rows = n // NS
    assert stage_rows * NS == n

    @functools.partial(
        pl.kernel,
        out_type=jax.ShapeDtypeStruct((NC, N_ACC, DH), jnp.bfloat16),
        mesh=_MESH,
        compiler_params=_SC_PARAMS,
        scratch_types=[
            pltpu.VMEM((k_chunks, CHUNK), jnp.int32),
            pltpu.VMEM((k_chunks, CHUNK), jnp.int32),
            pltpu.VMEM((2 * GROUP, CHUNK, DH), jnp.bfloat16),
            pltpu.VMEM_SHARED((N_ACC, DH), jnp.bfloat16),
            pltpu.VMEM_SHARED((n, DH), jnp.bfloat16),
            pltpu.SemaphoreType.DMA,
            pltpu.SemaphoreType.DMA,
            pltpu.SemaphoreType.DMA,
            pltpu.SemaphoreType.DMA,
        ],
    )
    def agg(h_hbm, zeros_hbm, srcs_hbm, dsts_hbm, out_hbm, src_v, dst_v,
            rows, acc_sh, h_sh, gsem0, gsem1, ssem, stsem):
        cid = lax.axis_index("c")
        sid = lax.axis_index("s")
        gsems = [gsem0, gsem1]

        sbase = sid * stage_rows
        stage_dma = pltpu.async_copy(
            h_hbm.at[pl.ds(sbase, stage_rows), pl.ds(cid * DH, DH)],
            h_sh.at[pl.ds(sbase, stage_rows)], stsem)

        pltpu.sync_copy(srcs_hbm.at[sid], src_v)
        pltpu.sync_copy(dsts_hbm.at[sid], dst_v)

        def gather(j, b, h):
            pltpu.async_copy(h_sh.at[src_v.at[j]], rows.at[b], gsems[h])

        def gather_wait(j, b, h):
            pltpu.make_async_copy(
                h_sh.at[src_v.at[j]], rows.at[b], gsems[h]).wait()

        def scatter(j, b):
            return pltpu.async_copy(
                rows.at[b], acc_sh.at[dst_v.at[j]], ssem, add=True)

        base = sid * rows_per_tile
        pltpu.sync_copy(zeros_hbm.at[pl.ds(base, rows_per_tile)],
                        acc_sh.at[pl.ds(base, rows_per_tile)])
        stage_dma.wait()
        plsc.subcore_barrier()

        _pipeline(k_chunks, gather, gather_wait, scatter)

        plsc.subcore_barrier()
        pltpu.sync_copy(acc_sh.at[pl.ds(base, rows_per_tile)],
                        out_hbm.at[cid, pl.ds(base, rows_per_tile)])

    return agg


def _scalar_agg_factory(n, k_chunks, with_gather):
    per_tile = N_ACC // NS
    assert per_tile % LANES == 0
    k_alloc = k_chunks

    scratch = [
        pltpu.VMEM((k_alloc, CHUNK), jnp.int32),
        pltpu.VMEM((k_chunks, CHUNK), jnp.int32),
        pltpu.VMEM((2 * GROUP, CHUNK), jnp.float32),
        pltpu.VMEM((per_tile,), jnp.float32),
        pltpu.VMEM_SHARED((N_ACC,), jnp.float32),
        pltpu.SemaphoreType.DMA,
        pltpu.SemaphoreType.DMA,
        pltpu.SemaphoreType.DMA,
    ]

    def body(vals_hbm, gidx_hbm, sidx_hbm, out_hbm, gidx_v, sidx_v, vals,
             zeros_v, acc_sh, gsem0, gsem1, ssem):
        cid = lax.axis_index("c")
        sid = lax.axis_index("s")
        wid = sid * NC + cid
        gsems = [gsem0, gsem1]

        if with_gather:
            pltpu.sync_copy(gidx_hbm.at[wid], gidx_v)
        pltpu.sync_copy(sidx_hbm.at[wid], sidx_v)

        def gather(j, b, h):
            pltpu.async_copy(vals_hbm.at[gidx_v.at[j]], vals.at[b], gsems[h])

        def gather_wait(j, b, h):
            pltpu.make_async_copy(
                vals_hbm.at[gidx_v.at[j]], vals.at[b], gsems[h]).wait()

        def scatter(j, b):
            return pltpu.async_copy(
                vals.at[b], acc_sh.at[sidx_v.at[j]], ssem, add=True)

        zf = jnp.zeros((LANES,), jnp.float32)

        @pl.loop(0, per_tile // LANES)
        def _(r):
            zeros_v[pl.ds(r * LANES, LANES)] = zf

        if not with_gather:
            one = jnp.ones((LANES,), jnp.float32)
            for b in range(2 * GROUP):
                for c in range(CHUNK // LANES):
                    vals[b, pl.ds(c * LANES, LANES)] = one

        base = sid * per_tile
        pltpu.sync_copy(zeros_v, acc_sh.at[pl.ds(base, per_tile)])
        plsc.subcore_barrier()

        if with_gather:
            _pipeline(k_chunks, gather, gather_wait, scatter)
        else:
            @pl.loop(0, k_chunks, step=2 * GROUP)
            def _(g):
                sds = [scatter(g + b, b) for b in range(2 * GROUP)]
                for dsc in sds:
                    dsc.wait()

        plsc.subcore_barrier()
        pltpu.sync_copy(acc_sh.at[pl.ds(base, per_tile)],
                        out_hbm.at[cid, pl.ds(base, per_tile)])

    return functools.partial(
        pl.kernel,
        out_type=jax.ShapeDtypeStruct((NC, N_ACC), jnp.float32),
        mesh=_MESH,
        compiler_params=_SC_PARAMS,
        scratch_types=scratch,
    )(body)


def _k1(degp3, x, w0, bn):
    n, d_in = x.shape
    d_h = w0.shape[1]
    grid = n // bn

    def body(deg_ref, x_ref, w_ref, h_ref, dis_ref):
        deg = deg_ref[0, :, 0] + deg_ref[1, :, 0] + 1.0
        dis = lax.rsqrt(deg)
        h = jnp.dot(x_ref[...], w_ref[...], preferred_element_type=jnp.float32)
        h_ref[...] = h * dis[:, None]
        dis_ref[...] = dis[:, None]

    return pl.pallas_call(
        body,
        grid=(grid,),
        in_specs=[
            pl.BlockSpec((NC, bn, 1), lambda i: (0, i, 0)),
            pl.BlockSpec((bn, d_in), lambda i: (i, 0)),
            pl.BlockSpec((d_in, d_h), lambda i: (0, 0)),
        ],
        out_specs=[
            pl.BlockSpec((bn, d_h), lambda i: (i, 0)),
            pl.BlockSpec((bn, 1), lambda i: (i, 0)),
        ],
        out_shape=[
            jax.ShapeDtypeStruct((n, d_h), jnp.float32),
            jax.ShapeDtypeStruct((n, 1), jnp.float32),
        ],
    )(degp3, x, w0)


def _k2(ap, hp, dis, b, w, bn):
    n, d = hp.shape
    d_out = w.shape[1]
    grid = n // bn

    def body(ap_ref, hp_ref, dis_ref, b_ref, w_ref, out_ref):
        agg = jnp.concatenate(
            [ap_ref[0], ap_ref[1]], axis=1).astype(jnp.float32)
        agg = agg + hp_ref[...]
        o = jnp.maximum(dis_ref[...] * agg + b_ref[...][None, :], 0.0)
        h = jnp.dot(o, w_ref[...], preferred_element_type=jnp.float32)
        out_ref[...] = h * dis_ref[...]

    return pl.pallas_call(
        body,
        grid=(grid,),
        in_specs=[
            pl.BlockSpec((NC, bn, DH), lambda i: (0, i, 0)),
            pl.BlockSpec((bn, d), lambda i: (i, 0)),
            pl.BlockSpec((bn, 1), lambda i: (i, 0)),
            pl.BlockSpec((d,), lambda i: (0,)),
            pl.BlockSpec((d, d_out), lambda i: (0, 0)),
        ],
        out_specs=pl.BlockSpec((bn, d_out), lambda i: (i, 0)),
        out_shape=jax.ShapeDtypeStruct((n, d_out), jnp.float32),
    )(ap, hp, dis, b, w)


def _k4(a2p3, h2p, dis, b2, batch2, n, g):
    def body(a2_ref, h2_ref, dis_ref, b2_ref, bat_ref, out_ref):
        a2 = a2_ref[0, :n, 0] + a2_ref[1, :n, 0]
        out2 = dis_ref[:, 0] * (a2 + h2_ref[:, 0]) + b2_ref[0]
        gid = bat_ref[:, 0]
        oh = (gid[:, None] == lax.broadcasted_iota(jnp.int32, (1, g), 1)
              ).astype(jnp.float32)
        sums = lax.dot_general(oh, out2[:, None],
                               (((0,), (0,)), ((), ())),
                               preferred_element_type=jnp.float32)
        counts = jnp.sum(oh, axis=0)
        mean = sums[:, 0] / jnp.maximum(counts, 1.0)
        out_ref[...] = 1.0 / (1.0 + jnp.exp(-mean))

    return pl.pallas_call(
        body,
        out_shape=jax.ShapeDtypeStruct((g,), jnp.float32),
    )(a2p3, h2p, dis, b2, batch2)


def kernel(x, edge_index, batch, W0, b0, W1, b1, W2, b2):
    n, d_in = x.shape
    e = edge_index.shape[1]
    g = 64
    bn = 2000

    k_chunks = -(-e // (NW * CHUNK))
    k_chunks = -(-k_chunks // (2 * GROUP)) * (2 * GROUP)
    e_pad = NW * k_chunks * CHUNK
    k2 = 2 * k_chunks
    src = edge_index[0]
    dst = edge_index[1]
    src_flat = jnp.concatenate([src, jnp.zeros((e_pad - e,), jnp.int32)])
    dst_flat = jnp.concatenate([dst, jnp.full((e_pad - e,), n, jnp.int32)])
    srcs = src_flat.reshape(NW, k_chunks, CHUNK)
    dsts = dst_flat.reshape(NW, k_chunks, CHUNK)
    srcs2 = src_flat.reshape(NS, k2, CHUNK)
    dsts2 = dst_flat.reshape(NS, k2, CHUNK)

    row_agg = _row_agg_factory(n, k2)
    scalar_agg = _scalar_agg_factory(n, k_chunks, True)
    deg_count = _scalar_agg_factory(n, k_chunks, False)

    ones_pad = jnp.ones((N_ACC,), jnp.float32)
    degp = deg_count(ones_pad, srcs, dsts)

    zeros_bf = jnp.zeros((N_ACC, DH), jnp.bfloat16)
    h0, dis = _k1(degp.reshape(NC, N_ACC, 1), x, W0, bn)
    a0 = row_agg(h0.astype(jnp.bfloat16), zeros_bf, srcs2, dsts2)
    h1 = _k2(a0, h0, dis, b0, W1, bn)
    a1 = row_agg(h1.astype(jnp.bfloat16), zeros_bf, srcs2, dsts2)
    h2p = _k2(a1, h1, dis, b1, W2, bn)

    h2pad = jnp.concatenate([h2p[:, 0], jnp.zeros((N_ACC - n,), jnp.float32)])
    a2 = scalar_agg(h2pad, srcs, dsts)

    return _k4(a2.reshape(NC, N_ACC, 1), h2p, dis, b2,
               batch.reshape(n, 1), n, g)


# speedup vs baseline: 7.0354x; 1.1405x over previous
import functools

import jax
import jax.numpy as jnp
from jax import lax
from jax.experimental import pallas as pl
from jax.experimental.pallas import tpu as pltpu
from jax.experimental.pallas import tpu_sc as plsc

NC = 2
NS = 16
NW = NC * NS
LANES = 16
CHUNK = 128
GROUP = 4
N_ACC = 10240
DH = 64
QW = 32

_MESH = plsc.VectorSubcoreMesh(core_axis_name="c", subcore_axis_name="s")
_SC_PARAMS = pltpu.CompilerParams(use_tc_tiling_on_sc=False)


def _pipeline(k_chunks, gather, gather_wait, scatter):
    nb = 2 * GROUP

    @pl.loop(0, k_chunks, step=nb)
    def _(g):
        for b in range(nb):
            gather(g + b, b, 0)
        for b in range(nb):
            gather_wait(g + b, b, 0)
        sds = [scatter(g + b, b) for b in range(nb)]
        for dsc in sds:
            dsc.wait()


def _row_agg_factory(n, k_chunks):
    rows_per_tile = N_ACC // NS
    assert rows_per_tile % 128 == 0
    stage_rows = n // NS
    assert stage_rows * NS == n

    @functools.partial(
        pl.kernel,
        out_type=jax.ShapeDtypeStruct((NC, N_ACC, DH), jnp.bfloat16),
        mesh=_MESH,
        compiler_params=_SC_PARAMS,
        scratch_types=[
            pltpu.VMEM((k_chunks, CHUNK), jnp.int32),
            pltpu.VMEM((k_chunks, CHUNK), jnp.int32),
            pltpu.VMEM((2 * GROUP, CHUNK, DH), jnp.bfloat16),
            pltpu.VMEM_SHARED((N_ACC, DH), jnp.bfloat16),
            pltpu.VMEM_SHARED((n, DH), jnp.bfloat16),
            pltpu.SemaphoreType.DMA,
            pltpu.SemaphoreType.DMA,
            pltpu.SemaphoreType.DMA,
            pltpu.SemaphoreType.DMA,
        ],
    )
    def agg(h_hbm, zeros_hbm, srcs_hbm, dsts_hbm, out_hbm, src_v, dst_v,
            rows, acc_sh, h_sh, gsem0, gsem1, ssem, stsem):
        cid = lax.axis_index("c")
        sid = lax.axis_index("s")
        gsems = [gsem0, gsem1]

        sbase = sid * stage_rows
        stage_dma = pltpu.async_copy(
            h_hbm.at[pl.ds(sbase, stage_rows), pl.ds(cid * DH, DH)],
            h_sh.at[pl.ds(sbase, stage_rows)], stsem)

        pltpu.sync_copy(srcs_hbm.at[sid], src_v)
        pltpu.sync_copy(dsts_hbm.at[sid], dst_v)

        def gather(j, b, h):
            pltpu.async_copy(h_sh.at[src_v.at[j]], rows.at[b], gsems[h])

        def gather_wait(j, b, h):
            pltpu.make_async_copy(
                h_sh.at[src_v.at[j]], rows.at[b], gsems[h]).wait()

        def scatter(j, b):
            return pltpu.async_copy(
                rows.at[b], acc_sh.at[dst_v.at[j]], ssem, add=True)

        base = sid * rows_per_tile
        pltpu.sync_copy(zeros_hbm.at[pl.ds(base, rows_per_tile)],
                        acc_sh.at[pl.ds(base, rows_per_tile)])
        stage_dma.wait()
        plsc.subcore_barrier()

        _pipeline(k_chunks, gather, gather_wait, scatter)

        plsc.subcore_barrier()
        pltpu.sync_copy(acc_sh.at[pl.ds(base, rows_per_tile)],
                        out_hbm.at[cid, pl.ds(base, rows_per_tile)])

    return agg


def _scalar_agg_factory(n, k_chunks, with_gather):
    per_tile = N_ACC // NS
    assert per_tile % LANES == 0
    k_alloc = k_chunks

    scratch = [
        pltpu.VMEM((k_alloc, CHUNK), jnp.int32),
        pltpu.VMEM((k_chunks, CHUNK), jnp.int32),
        pltpu.VMEM((2 * GROUP, CHUNK), jnp.float32),
        pltpu.VMEM((per_tile,), jnp.float32),
        pltpu.VMEM_SHARED((N_ACC,), jnp.float32),
        pltpu.VMEM_SHARED((N_ACC,), jnp.float32),
        pltpu.SemaphoreType.DMA,
        pltpu.SemaphoreType.DMA,
        pltpu.SemaphoreType.DMA,
        pltpu.SemaphoreType.DMA,
    ]

    def body(vals_hbm, gidx_hbm, sidx_hbm, out_hbm, gidx_v, sidx_v, vals,
             zeros_v, acc_sh, v_sh, gsem0, gsem1, ssem, stsem):
        cid = lax.axis_index("c")
        sid = lax.axis_index("s")
        wid = sid * NC + cid
        gsems = [gsem0, gsem1]

        stage_dma = None
        if with_gather:
            stage_dma = pltpu.async_copy(
                vals_hbm.at[pl.ds(sid * per_tile, per_tile)],
                v_sh.at[pl.ds(sid * per_tile, per_tile)], stsem)
            pltpu.sync_copy(gidx_hbm.at[wid], gidx_v)
        pltpu.sync_copy(sidx_hbm.at[wid], sidx_v)

        def gather(j, b, h):
            pltpu.async_copy(v_sh.at[gidx_v.at[j]], vals.at[b], gsems[h])

        def gather_wait(j, b, h):
            pltpu.make_async_copy(
                v_sh.at[gidx_v.at[j]], vals.at[b], gsems[h]).wait()

        def scatter(j, b):
            return pltpu.async_copy(
                vals.at[b], acc_sh.at[sidx_v.at[j]], ssem, add=True)

        zf = jnp.zeros((LANES,), jnp.float32)

        @pl.loop(0, per_tile // LANES)
        def _(r):
            zeros_v[pl.ds(r * LANES, LANES)] = zf

        if not with_gather:
            one = jnp.ones((LANES,), jnp.float32)
            for b in range(2 * GROUP):
                for c in range(CHUNK // LANES):
                    vals[b, pl.ds(c * LANES, LANES)] = one

        base = sid * per_tile
        pltpu.sync_copy(zeros_v, acc_sh.at[pl.ds(base, per_tile)])
        if stage_dma is not None:
            stage_dma.wait()
        plsc.subcore_barrier()

        if with_gather:
            _pipeline(k_chunks, gather, gather_wait, scatter)
        else:
            @pl.loop(0, k_chunks, step=2 * GROUP)
            def _(g):
                sds = [scatter(g + b, b) for b in range(2 * GROUP)]
                for dsc in sds:
                    dsc.wait()

        plsc.subcore_barrier()
        pltpu.sync_copy(acc_sh.at[pl.ds(base, per_tile)],
                        out_hbm.at[cid, pl.ds(base, per_tile)])

    return functools.partial(
        pl.kernel,
        out_type=jax.ShapeDtypeStruct((NC, N_ACC), jnp.float32),
        mesh=_MESH,
        compiler_params=_SC_PARAMS,
        scratch_types=scratch,
    )(body)


def _k1(degp3, x, w0, bn):
    n, d_in = x.shape
    d_h = w0.shape[1]
    grid = n // bn

    def body(deg_ref, x_ref, w_ref, h_ref, dis_ref):
        deg = deg_ref[0, :, 0] + deg_ref[1, :, 0] + 1.0
        dis = lax.rsqrt(deg)
        h = jnp.dot(x_ref[...], w_ref[...], preferred_element_type=jnp.float32)
        h_ref[...] = h * dis[:, None]
        dis_ref[...] = dis[:, None]

    return pl.pallas_call(
        body,
        grid=(grid,),
        in_specs=[
            pl.BlockSpec((NC, bn, 1), lambda i: (0, i, 0)),
            pl.BlockSpec((bn, d_in), lambda i: (i, 0)),
            pl.BlockSpec((d_in, d_h), lambda i: (0, 0)),
        ],
        out_specs=[
            pl.BlockSpec((bn, d_h), lambda i: (i, 0)),
            pl.BlockSpec((bn, 1), lambda i: (i, 0)),
        ],
        out_shape=[
            jax.ShapeDtypeStruct((n, d_h), jnp.float32),
            jax.ShapeDtypeStruct((n, 1), jnp.float32),
        ],
    )(degp3, x, w0)


def _k2(ap, hp, dis, b, w, bn):
    n, d = hp.shape
    d_out = w.shape[1]
    grid = n // bn

    def body(ap_ref, hp_ref, dis_ref, b_ref, w_ref, out_ref):
        agg = jnp.concatenate(
            [ap_ref[0], ap_ref[1]], axis=1).astype(jnp.float32)
        agg = agg + hp_ref[...]
        o = jnp.maximum(dis_ref[...] * agg + b_ref[...][None, :], 0.0)
        h = jnp.dot(o, w_ref[...], preferred_element_type=jnp.float32)
        out_ref[...] = h * dis_ref[...]

    return pl.pallas_call(
        body,
        grid=(grid,),
        in_specs=[
            pl.BlockSpec((NC, bn, DH), lambda i: (0, i, 0)),
            pl.BlockSpec((bn, d), lambda i: (i, 0)),
            pl.BlockSpec((bn, 1), lambda i: (i, 0)),
            pl.BlockSpec((d,), lambda i: (0,)),
            pl.BlockSpec((d, d_out), lambda i: (0, 0)),
        ],
        out_specs=pl.BlockSpec((bn, d_out), lambda i: (i, 0)),
        out_shape=jax.ShapeDtypeStruct((n, d_out), jnp.float32),
    )(ap, hp, dis, b, w)


def _k4(a2p3, h2p, dis, b2, batch2, n, g):
    def body(a2_ref, h2_ref, dis_ref, b2_ref, bat_ref, out_ref):
        a2 = a2_ref[0, :n, 0] + a2_ref[1, :n, 0]
        out2 = dis_ref[:, 0] * (a2 + h2_ref[:, 0]) + b2_ref[0]
        gid = bat_ref[:, 0]
        oh = (gid[:, None] == lax.broadcasted_iota(jnp.int32, (1, g), 1)
              ).astype(jnp.float32)
        sums = lax.dot_general(oh, out2[:, None],
                               (((0,), (0,)), ((), ())),
                               preferred_element_type=jnp.float32)
        counts = jnp.sum(oh, axis=0)
        mean = sums[:, 0] / jnp.maximum(counts, 1.0)
        out_ref[...] = 1.0 / (1.0 + jnp.exp(-mean))

    return pl.pallas_call(
        body,
        out_shape=jax.ShapeDtypeStruct((g,), jnp.float32),
    )(a2p3, h2p, dis, b2, batch2)


def kernel(x, edge_index, batch, W0, b0, W1, b1, W2, b2):
    n, d_in = x.shape
    e = edge_index.shape[1]
    g = 64
    bn = 2000

    k_chunks = -(-e // (NW * CHUNK))
    k_chunks = -(-k_chunks // (2 * GROUP)) * (2 * GROUP)
    e_pad = NW * k_chunks * CHUNK
    k2 = 2 * k_chunks
    src = edge_index[0]
    dst = edge_index[1]
    src_flat = jnp.concatenate([src, jnp.zeros((e_pad - e,), jnp.int32)])
    dst_flat = jnp.concatenate([dst, jnp.full((e_pad - e,), n, jnp.int32)])
    srcs = src_flat.reshape(NW, k_chunks, CHUNK)
    dsts = dst_flat.reshape(NW, k_chunks, CHUNK)
    srcs2 = src_flat.reshape(NS, k2, CHUNK)
    dsts2 = dst_flat.reshape(NS, k2, CHUNK)

    row_agg = _row_agg_factory(n, k2)
    scalar_agg = _scalar_agg_factory(n, k_chunks, True)
    deg_count = _scalar_agg_factory(n, k_chunks, False)

    ones_pad = jnp.ones((N_ACC,), jnp.float32)
    degp = deg_count(ones_pad, srcs, dsts)

    zeros_bf = jnp.zeros((N_ACC, DH), jnp.bfloat16)
    h0, dis = _k1(degp.reshape(NC, N_ACC, 1), x, W0, bn)
    a0 = row_agg(h0.astype(jnp.bfloat16), zeros_bf, srcs2, dsts2)
    h1 = _k2(a0, h0, dis, b0, W1, bn)
    a1 = row_agg(h1.astype(jnp.bfloat16), zeros_bf, srcs2, dsts2)
    h2p = _k2(a1, h1, dis, b1, W2, bn)

    h2pad = jnp.concatenate([h2p[:, 0], jnp.zeros((N_ACC - n,), jnp.float32)])
    a2 = scalar_agg(h2pad, srcs, dsts)

    return _k4(a2.reshape(NC, N_ACC, 1), h2p, dis, b2,
               batch.reshape(n, 1), n, g)


# speedup vs baseline: 7.7512x; 1.1018x over previous
import functools

import jax
import jax.numpy as jnp
from jax import lax
from jax.experimental import pallas as pl
from jax.experimental.pallas import tpu as pltpu
from jax.experimental.pallas import tpu_sc as plsc

NC = 2
NS = 16
NW = NC * NS
LANES = 16
CHUNK = 128
GROUP = 4
N_ACC = 10240
DH = 64
QW = 32

_MESH = plsc.VectorSubcoreMesh(core_axis_name="c", subcore_axis_name="s")
_SC_PARAMS = pltpu.CompilerParams(use_tc_tiling_on_sc=False)


def _pipeline(k_chunks, gather, gather_wait, scatter):
    nh = GROUP
    n_groups = k_chunks // nh
    assert n_groups >= 2 and n_groups % 2 == 0

    def fire(g, h):
        for b in range(nh):
            gather(g * nh + b, h * nh + b, h)

    def wait_g(g, h):
        for b in range(nh):
            gather_wait(g * nh + b, h * nh + b, h)

    def scat(g, h):
        sds = [scatter(g * nh + b, h * nh + b) for b in range(nh)]
        for dsc in sds:
            dsc.wait()

    fire(0, 0)
    if n_groups > 2:

        @pl.loop(0, n_groups - 2, step=2)
        def _(g):
            fire(g + 1, 1)
            wait_g(g, 0)
            scat(g, 0)
            fire(g + 2, 0)
            wait_g(g + 1, 1)
            scat(g + 1, 1)

    fire(n_groups - 1, 1)
    wait_g(n_groups - 2, 0)
    scat(n_groups - 2, 0)
    wait_g(n_groups - 1, 1)
    scat(n_groups - 1, 1)


def _row_agg_factory(n, k_chunks):
    rows_per_tile = N_ACC // NS
    assert rows_per_tile % 128 == 0
    stage_rows = n // NS
    assert stage_rows * NS == n

    @functools.partial(
        pl.kernel,
        out_type=jax.ShapeDtypeStruct((NC, N_ACC, DH), jnp.bfloat16),
        mesh=_MESH,
        compiler_params=_SC_PARAMS,
        scratch_types=[
            pltpu.VMEM((k_chunks, CHUNK), jnp.int32),
            pltpu.VMEM((k_chunks, CHUNK), jnp.int32),
            pltpu.VMEM((2 * GROUP, CHUNK, DH), jnp.bfloat16),
            pltpu.VMEM_SHARED((N_ACC, DH), jnp.bfloat16),
            pltpu.VMEM_SHARED((n, DH), jnp.bfloat16),
            pltpu.SemaphoreType.DMA,
            pltpu.SemaphoreType.DMA,
            pltpu.SemaphoreType.DMA,
            pltpu.SemaphoreType.DMA,
        ],
    )
    def agg(h_hbm, zeros_hbm, srcs_hbm, dsts_hbm, out_hbm, src_v, dst_v,
            rows, acc_sh, h_sh, gsem0, gsem1, ssem, stsem):
        cid = lax.axis_index("c")
        sid = lax.axis_index("s")
        gsems = [gsem0, gsem1]

        sbase = sid * stage_rows
        stage_dma = pltpu.async_copy(
            h_hbm.at[pl.ds(sbase, stage_rows), pl.ds(cid * DH, DH)],
            h_sh.at[pl.ds(sbase, stage_rows)], stsem)

        pltpu.sync_copy(srcs_hbm.at[sid], src_v)
        pltpu.sync_copy(dsts_hbm.at[sid], dst_v)

        def gather(j, b, h):
            pltpu.async_copy(h_sh.at[src_v.at[j]], rows.at[b], gsems[h])

        def gather_wait(j, b, h):
            pltpu.make_async_copy(
                h_sh.at[src_v.at[j]], rows.at[b], gsems[h]).wait()

        def scatter(j, b):
            return pltpu.async_copy(
                rows.at[b], acc_sh.at[dst_v.at[j]], ssem, add=True)

        base = sid * rows_per_tile
        pltpu.sync_copy(zeros_hbm.at[pl.ds(base, rows_per_tile)],
                        acc_sh.at[pl.ds(base, rows_per_tile)])
        stage_dma.wait()
        plsc.subcore_barrier()

        _pipeline(k_chunks, gather, gather_wait, scatter)

        plsc.subcore_barrier()
        pltpu.sync_copy(acc_sh.at[pl.ds(base, rows_per_tile)],
                        out_hbm.at[cid, pl.ds(base, rows_per_tile)])

    return agg


def _scalar_agg_factory(n, k_chunks, with_gather):
    per_tile = N_ACC // NS
    assert per_tile % LANES == 0
    k_alloc = k_chunks

    scratch = [
        pltpu.VMEM((k_alloc, CHUNK), jnp.int32),
        pltpu.VMEM((k_chunks, CHUNK), jnp.int32),
        pltpu.VMEM((2 * GROUP, CHUNK), jnp.float32),
        pltpu.VMEM((per_tile,), jnp.float32),
        pltpu.VMEM_SHARED((N_ACC,), jnp.float32),
        pltpu.VMEM_SHARED((N_ACC,), jnp.float32),
        pltpu.SemaphoreType.DMA,
        pltpu.SemaphoreType.DMA,
        pltpu.SemaphoreType.DMA,
        pltpu.SemaphoreType.DMA,
    ]

    def body(vals_hbm, gidx_hbm, sidx_hbm, out_hbm, gidx_v, sidx_v, vals,
             zeros_v, acc_sh, v_sh, gsem0, gsem1, ssem, stsem):
        cid = lax.axis_index("c")
        sid = lax.axis_index("s")
        wid = sid * NC + cid
        gsems = [gsem0, gsem1]

        stage_dma = None
        if with_gather:
            stage_dma = pltpu.async_copy(
                vals_hbm.at[pl.ds(sid * per_tile, per_tile)],
                v_sh.at[pl.ds(sid * per_tile, per_tile)], stsem)
            pltpu.sync_copy(gidx_hbm.at[wid], gidx_v)
        pltpu.sync_copy(sidx_hbm.at[wid], sidx_v)

        def gather(j, b, h):
            pltpu.async_copy(v_sh.at[gidx_v.at[j]], vals.at[b], gsems[h])

        def gather_wait(j, b, h):
            pltpu.make_async_copy(
                v_sh.at[gidx_v.at[j]], vals.at[b], gsems[h]).wait()

        def scatter(j, b):
            return pltpu.async_copy(
                vals.at[b], acc_sh.at[sidx_v.at[j]], ssem, add=True)

        zf = jnp.zeros((LANES,), jnp.float32)

        @pl.loop(0, per_tile // LANES)
        def _(r):
            zeros_v[pl.ds(r * LANES, LANES)] = zf

        if not with_gather:
            one = jnp.ones((LANES,), jnp.float32)
            for b in range(2 * GROUP):
                for c in range(CHUNK // LANES):
                    vals[b, pl.ds(c * LANES, LANES)] = one

        base = sid * per_tile
        pltpu.sync_copy(zeros_v, acc_sh.at[pl.ds(base, per_tile)])
        if stage_dma is not None:
            stage_dma.wait()
        plsc.subcore_barrier()

        if with_gather:
            _pipeline(k_chunks, gather, gather_wait, scatter)
        else:
            @pl.loop(0, k_chunks, step=2 * GROUP)
            def _(g):
                sds = [scatter(g + b, b) for b in range(2 * GROUP)]
                for dsc in sds:
                    dsc.wait()

        plsc.subcore_barrier()
        pltpu.sync_copy(acc_sh.at[pl.ds(base, per_tile)],
                        out_hbm.at[cid, pl.ds(base, per_tile)])

    return functools.partial(
        pl.kernel,
        out_type=jax.ShapeDtypeStruct((NC, N_ACC), jnp.float32),
        mesh=_MESH,
        compiler_params=_SC_PARAMS,
        scratch_types=scratch,
    )(body)


def _k1(degp3, x, w0, bn):
    n, d_in = x.shape
    d_h = w0.shape[1]
    grid = n // bn

    def body(deg_ref, x_ref, w_ref, h_ref, dis_ref):
        deg = deg_ref[0, :, 0] + deg_ref[1, :, 0] + 1.0
        dis = lax.rsqrt(deg)
        h = jnp.dot(x_ref[...], w_ref[...], preferred_element_type=jnp.float32)
        h_ref[...] = h * dis[:, None]
        dis_ref[...] = dis[:, None]

    return pl.pallas_call(
        body,
        grid=(grid,),
        in_specs=[
            pl.BlockSpec((NC, bn, 1), lambda i: (0, i, 0)),
            pl.BlockSpec((bn, d_in), lambda i: (i, 0)),
            pl.BlockSpec((d_in, d_h), lambda i: (0, 0)),
        ],
        out_specs=[
            pl.BlockSpec((bn, d_h), lambda i: (i, 0)),
            pl.BlockSpec((bn, 1), lambda i: (i, 0)),
        ],
        out_shape=[
            jax.ShapeDtypeStruct((n, d_h), jnp.float32),
            jax.ShapeDtypeStruct((n, 1), jnp.float32),
        ],
    )(degp3, x, w0)


def _k2(ap, hp, dis, b, w, bn):
    n, d = hp.shape
    d_out = w.shape[1]
    grid = n // bn

    def body(ap_ref, hp_ref, dis_ref, b_ref, w_ref, out_ref):
        agg = jnp.concatenate(
            [ap_ref[0], ap_ref[1]], axis=1).astype(jnp.float32)
        agg = agg + hp_ref[...]
        o = jnp.maximum(dis_ref[...] * agg + b_ref[...][None, :], 0.0)
        h = jnp.dot(o, w_ref[...], preferred_element_type=jnp.float32)
        out_ref[...] = h * dis_ref[...]

    return pl.pallas_call(
        body,
        grid=(grid,),
        in_specs=[
            pl.BlockSpec((NC, bn, DH), lambda i: (0, i, 0)),
            pl.BlockSpec((bn, d), lambda i: (i, 0)),
            pl.BlockSpec((bn, 1), lambda i: (i, 0)),
            pl.BlockSpec((d,), lambda i: (0,)),
            pl.BlockSpec((d, d_out), lambda i: (0, 0)),
        ],
        out_specs=pl.BlockSpec((bn, d_out), lambda i: (i, 0)),
        out_shape=jax.ShapeDtypeStruct((n, d_out), jnp.float32),
    )(ap, hp, dis, b, w)


def _k4(a2p3, h2p, dis, b2, batch2, n, g):
    def body(a2_ref, h2_ref, dis_ref, b2_ref, bat_ref, out_ref):
        a2 = a2_ref[0, :n, 0] + a2_ref[1, :n, 0]
        out2 = dis_ref[:, 0] * (a2 + h2_ref[:, 0]) + b2_ref[0]
        gid = bat_ref[:, 0]
        oh = (gid[:, None] == lax.broadcasted_iota(jnp.int32, (1, g), 1)
              ).astype(jnp.float32)
        sums = lax.dot_general(oh, out2[:, None],
                               (((0,), (0,)), ((), ())),
                               preferred_element_type=jnp.float32)
        counts = jnp.sum(oh, axis=0)
        mean = sums[:, 0] / jnp.maximum(counts, 1.0)
        out_ref[...] = 1.0 / (1.0 + jnp.exp(-mean))

    return pl.pallas_call(
        body,
        out_shape=jax.ShapeDtypeStruct((g,), jnp.float32),
    )(a2p3, h2p, dis, b2, batch2)


def kernel(x, edge_index, batch, W0, b0, W1, b1, W2, b2):
    n, d_in = x.shape
    e = edge_index.shape[1]
    g = 64
    bn = 2000

    k_chunks = -(-e // (NW * CHUNK))
    k_chunks = -(-k_chunks // (2 * GROUP)) * (2 * GROUP)
    e_pad = NW * k_chunks * CHUNK
    k2 = 2 * k_chunks
    src = edge_index[0]
    dst = edge_index[1]
    src_flat = jnp.concatenate([src, jnp.zeros((e_pad - e,), jnp.int32)])
    dst_flat = jnp.concatenate([dst, jnp.full((e_pad - e,), n, jnp.int32)])
    srcs = src_flat.reshape(NW, k_chunks, CHUNK)
    dsts = dst_flat.reshape(NW, k_chunks, CHUNK)
    srcs2 = src_flat.reshape(NS, k2, CHUNK)
    dsts2 = dst_flat.reshape(NS, k2, CHUNK)

    row_agg = _row_agg_factory(n, k2)
    scalar_agg = _scalar_agg_factory(n, k_chunks, True)
    deg_count = _scalar_agg_factory(n, k_chunks, False)

    ones_pad = jnp.ones((N_ACC,), jnp.float32)
    degp = deg_count(ones_pad, srcs, dsts)

    zeros_bf = jnp.zeros((N_ACC, DH), jnp.bfloat16)
    h0, dis = _k1(degp.reshape(NC, N_ACC, 1), x, W0, bn)
    a0 = row_agg(h0.astype(jnp.bfloat16), zeros_bf, srcs2, dsts2)
    h1 = _k2(a0, h0, dis, b0, W1, bn)
    a1 = row_agg(h1.astype(jnp.bfloat16), zeros_bf, srcs2, dsts2)
    h2p = _k2(a1, h1, dis, b1, W2, bn)

    h2pad = jnp.concatenate([h2p[:, 0], jnp.zeros((N_ACC - n,), jnp.float32)])
    a2 = scalar_agg(h2pad, srcs, dsts)

    return _k4(a2.reshape(NC, N_ACC, 1), h2p, dis, b2,
               batch.reshape(n, 1), n, g)


# speedup vs baseline: 7.8395x; 1.0114x over previous
import functools

import jax
import jax.numpy as jnp
from jax import lax
from jax.experimental import pallas as pl
from jax.experimental.pallas import tpu as pltpu
from jax.experimental.pallas import tpu_sc as plsc

NC = 2
NS = 16
NW = NC * NS
LANES = 16
CHUNK = 128
GROUP = 4
N_ACC = 10240
DH = 64
QW = 32

_MESH = plsc.VectorSubcoreMesh(core_axis_name="c", subcore_axis_name="s")
_SC_PARAMS = pltpu.CompilerParams(use_tc_tiling_on_sc=False)


def _pipeline(k_chunks, gather, gather_wait, scatter):
    nh = GROUP
    n_groups = k_chunks // nh
    assert n_groups >= 2 and n_groups % 2 == 0

    def fire(g, h):
        for b in range(nh):
            gather(g * nh + b, h * nh + b, h)

    def wait_g(g, h):
        for b in range(nh):
            gather_wait(g * nh + b, h * nh + b, h)

    def scat(g, h):
        sds = [scatter(g * nh + b, h * nh + b) for b in range(nh)]
        for dsc in sds:
            dsc.wait()

    fire(0, 0)
    if n_groups > 2:

        @pl.loop(0, n_groups - 2, step=2)
        def _(g):
            fire(g + 1, 1)
            wait_g(g, 0)
            scat(g, 0)
            fire(g + 2, 0)
            wait_g(g + 1, 1)
            scat(g + 1, 1)

    fire(n_groups - 1, 1)
    wait_g(n_groups - 2, 0)
    scat(n_groups - 2, 0)
    wait_g(n_groups - 1, 1)
    scat(n_groups - 1, 1)


def _row_agg_factory(n, k_chunks):
    rows_per_tile = N_ACC // NS
    assert rows_per_tile % 128 == 0
    stage_rows = n // NS
    assert stage_rows * NS == n

    @functools.partial(
        pl.kernel,
        out_type=jax.ShapeDtypeStruct((NC, N_ACC, DH), jnp.bfloat16),
        mesh=_MESH,
        compiler_params=_SC_PARAMS,
        scratch_types=[
            pltpu.VMEM((k_chunks, CHUNK), jnp.int32),
            pltpu.VMEM((k_chunks, CHUNK), jnp.int32),
            pltpu.VMEM((2 * GROUP, CHUNK, DH), jnp.bfloat16),
            pltpu.VMEM_SHARED((N_ACC, DH), jnp.bfloat16),
            pltpu.VMEM_SHARED((n, DH), jnp.bfloat16),
            pltpu.SemaphoreType.DMA,
            pltpu.SemaphoreType.DMA,
            pltpu.SemaphoreType.DMA,
            pltpu.SemaphoreType.DMA,
        ],
    )
    def agg(h_hbm, zeros_hbm, srcs_hbm, dsts_hbm, out_hbm, src_v, dst_v,
            rows, acc_sh, h_sh, gsem0, gsem1, ssem, stsem):
        cid = lax.axis_index("c")
        sid = lax.axis_index("s")
        gsems = [gsem0, gsem1]

        sbase = sid * stage_rows
        stage_dma = pltpu.async_copy(
            h_hbm.at[pl.ds(sbase, stage_rows), pl.ds(cid * DH, DH)],
            h_sh.at[pl.ds(sbase, stage_rows)], stsem)

        pltpu.sync_copy(srcs_hbm.at[sid], src_v)
        pltpu.sync_copy(dsts_hbm.at[sid], dst_v)

        def gather(j, b, h):
            pltpu.async_copy(h_sh.at[src_v.at[j]], rows.at[b], gsems[h])

        def gather_wait(j, b, h):
            pltpu.make_async_copy(
                h_sh.at[src_v.at[j]], rows.at[b], gsems[h]).wait()

        def scatter(j, b):
            return pltpu.async_copy(
                rows.at[b], acc_sh.at[dst_v.at[j]], ssem, add=True)

        base = sid * rows_per_tile
        pltpu.sync_copy(zeros_hbm.at[pl.ds(base, rows_per_tile)],
                        acc_sh.at[pl.ds(base, rows_per_tile)])
        stage_dma.wait()
        plsc.subcore_barrier()

        _pipeline(k_chunks, gather, gather_wait, scatter)

        plsc.subcore_barrier()
        pltpu.sync_copy(acc_sh.at[pl.ds(base, rows_per_tile)],
                        out_hbm.at[cid, pl.ds(base, rows_per_tile)])

    return agg


def _scalar_agg_factory(n, k_chunks, with_gather):
    per_tile = N_ACC // NS
    assert per_tile % LANES == 0
    k_alloc = k_chunks

    scratch = [
        pltpu.VMEM((k_alloc, CHUNK), jnp.int32),
        pltpu.VMEM((k_chunks, CHUNK), jnp.int32),
        pltpu.VMEM((2 * GROUP, CHUNK), jnp.float32),
        pltpu.VMEM((per_tile,), jnp.float32),
        pltpu.VMEM_SHARED((N_ACC,), jnp.float32),
        pltpu.VMEM_SHARED((N_ACC,), jnp.float32),
        pltpu.SemaphoreType.DMA,
        pltpu.SemaphoreType.DMA,
        pltpu.SemaphoreType.DMA,
        pltpu.SemaphoreType.DMA,
    ]

    def body(vals_hbm, gidx_hbm, sidx_hbm, out_hbm, gidx_v, sidx_v, vals,
             zeros_v, acc_sh, v_sh, gsem0, gsem1, ssem, stsem):
        cid = lax.axis_index("c")
        sid = lax.axis_index("s")
        wid = sid * NC + cid
        gsems = [gsem0, gsem1]

        stage_dma = None
        if with_gather:
            stage_dma = pltpu.async_copy(
                vals_hbm.at[pl.ds(sid * per_tile, per_tile)],
                v_sh.at[pl.ds(sid * per_tile, per_tile)], stsem)
            pltpu.sync_copy(gidx_hbm.at[wid], gidx_v)
        pltpu.sync_copy(sidx_hbm.at[wid], sidx_v)

        def gather(j, b, h):
            pltpu.async_copy(v_sh.at[gidx_v.at[j]], vals.at[b], gsems[h])

        def gather_wait(j, b, h):
            pltpu.make_async_copy(
                v_sh.at[gidx_v.at[j]], vals.at[b], gsems[h]).wait()

        def scatter(j, b):
            return pltpu.async_copy(
                vals.at[b], acc_sh.at[sidx_v.at[j]], ssem, add=True)

        zf = jnp.zeros((LANES,), jnp.float32)

        @pl.loop(0, per_tile // LANES)
        def _(r):
            zeros_v[pl.ds(r * LANES, LANES)] = zf

        if not with_gather:
            one = jnp.ones((LANES,), jnp.float32)
            for b in range(2 * GROUP):
                for c in range(CHUNK // LANES):
                    vals[b, pl.ds(c * LANES, LANES)] = one

        base = sid * per_tile
        pltpu.sync_copy(zeros_v, acc_sh.at[pl.ds(base, per_tile)])
        if stage_dma is not None:
            stage_dma.wait()
        plsc.subcore_barrier()

        if with_gather:
            _pipeline(k_chunks, gather, gather_wait, scatter)
        else:
            @pl.loop(0, k_chunks, step=2 * GROUP)
            def _(g):
                sds = [scatter(g + b, b) for b in range(2 * GROUP)]
                for dsc in sds:
                    dsc.wait()

        plsc.subcore_barrier()
        pltpu.sync_copy(acc_sh.at[pl.ds(base, per_tile)],
                        out_hbm.at[cid, pl.ds(base, per_tile)])

    return functools.partial(
        pl.kernel,
        out_type=jax.ShapeDtypeStruct((NC, N_ACC), jnp.float32),
        mesh=_MESH,
        compiler_params=_SC_PARAMS,
        scratch_types=scratch,
    )(body)


def _k1(degp3, x, w0, bn):
    n, d_in = x.shape
    d_h = w0.shape[1]
    grid = n // bn

    def body(deg_ref, x_ref, w_ref, h_ref, dis_ref):
        deg = deg_ref[0, :, 0] + deg_ref[1, :, 0] + 1.0
        dis = lax.rsqrt(deg)
        h = jnp.dot(x_ref[...], w_ref[...], preferred_element_type=jnp.float32)
        h_ref[...] = h * dis[:, None]
        dis_ref[...] = dis[:, None]

    return pl.pallas_call(
        body,
        grid=(grid,),
        in_specs=[
            pl.BlockSpec((NC, bn, 1), lambda i: (0, i, 0)),
            pl.BlockSpec((bn, d_in), lambda i: (i, 0)),
            pl.BlockSpec((d_in, d_h), lambda i: (0, 0)),
        ],
        out_specs=[
            pl.BlockSpec((bn, d_h), lambda i: (i, 0)),
            pl.BlockSpec((bn, 1), lambda i: (i, 0)),
        ],
        out_shape=[
            jax.ShapeDtypeStruct((n, d_h), jnp.float32),
            jax.ShapeDtypeStruct((n, 1), jnp.float32),
        ],
    )(degp3, x, w0)


def _k2(ap, hp, dis, b, w, bn):
    n, d = hp.shape
    d_out = w.shape[1]
    grid = n // bn

    def body(ap_ref, hp_ref, dis_ref, b_ref, w_ref, out_ref):
        agg = jnp.concatenate(
            [ap_ref[0], ap_ref[1]], axis=1).astype(jnp.float32)
        agg = agg + hp_ref[...]
        o = jnp.maximum(dis_ref[...] * agg + b_ref[...][None, :], 0.0)
        h = jnp.dot(o, w_ref[...], preferred_element_type=jnp.float32)
        out_ref[...] = h * dis_ref[...]

    return pl.pallas_call(
        body,
        grid=(grid,),
        in_specs=[
            pl.BlockSpec((NC, bn, DH), lambda i: (0, i, 0)),
            pl.BlockSpec((bn, d), lambda i: (i, 0)),
            pl.BlockSpec((bn, 1), lambda i: (i, 0)),
            pl.BlockSpec((d,), lambda i: (0,)),
            pl.BlockSpec((d, d_out), lambda i: (0, 0)),
        ],
        out_specs=pl.BlockSpec((bn, d_out), lambda i: (i, 0)),
        out_shape=jax.ShapeDtypeStruct((n, d_out), jnp.float32),
    )(ap, hp, dis, b, w)


def _k4(a2p3, h2p, dis, b2, batch2, n, g):
    def body(a2_ref, h2_ref, dis_ref, b2_ref, bat_ref, out_ref):
        a2 = a2_ref[0, :n, 0] + a2_ref[1, :n, 0]
        out2 = dis_ref[:, 0] * (a2 + h2_ref[:, 0]) + b2_ref[0]
        gid = bat_ref[:, 0]
        oh = (gid[:, None] == lax.broadcasted_iota(jnp.int32, (1, g), 1)
              ).astype(jnp.float32)
        sums = lax.dot_general(oh, out2[:, None],
                               (((0,), (0,)), ((), ())),
                               preferred_element_type=jnp.float32)
        counts = jnp.sum(oh, axis=0)
        mean = sums[:, 0] / jnp.maximum(counts, 1.0)
        out_ref[...] = 1.0 / (1.0 + jnp.exp(-mean))

    return pl.pallas_call(
        body,
        out_shape=jax.ShapeDtypeStruct((g,), jnp.float32),
    )(a2p3, h2p, dis, b2, batch2)


def kernel(x, edge_index, batch, W0, b0, W1, b1, W2, b2):
    n, d_in = x.shape
    e = edge_index.shape[1]
    g = 64
    bn = 5000

    k_chunks = -(-e // (NW * CHUNK))
    k_chunks = -(-k_chunks // (2 * GROUP)) * (2 * GROUP)
    e_pad = NW * k_chunks * CHUNK
    k2 = 2 * k_chunks
    src = edge_index[0]
    dst = edge_index[1]
    src_flat = jnp.concatenate([src, jnp.zeros((e_pad - e,), jnp.int32)])
    dst_flat = jnp.concatenate([dst, jnp.full((e_pad - e,), n, jnp.int32)])
    srcs = src_flat.reshape(NW, k_chunks, CHUNK)
    dsts = dst_flat.reshape(NW, k_chunks, CHUNK)
    srcs2 = src_flat.reshape(NS, k2, CHUNK)
    dsts2 = dst_flat.reshape(NS, k2, CHUNK)

    row_agg = _row_agg_factory(n, k2)
    scalar_agg = _scalar_agg_factory(n, k_chunks, True)
    deg_count = _scalar_agg_factory(n, k_chunks, False)

    ones_pad = jnp.ones((N_ACC,), jnp.float32)
    degp = deg_count(ones_pad, srcs, dsts)

    zeros_bf = jnp.zeros((N_ACC, DH), jnp.bfloat16)
    h0, dis = _k1(degp.reshape(NC, N_ACC, 1), x, W0, bn)
    a0 = row_agg(h0.astype(jnp.bfloat16), zeros_bf, srcs2, dsts2)
    h1 = _k2(a0, h0, dis, b0, W1, bn)
    a1 = row_agg(h1.astype(jnp.bfloat16), zeros_bf, srcs2, dsts2)
    h2p = _k2(a1, h1, dis, b1, W2, bn)

    h2pad = jnp.concatenate([h2p[:, 0], jnp.zeros((N_ACC - n,), jnp.float32)])
    a2 = scalar_agg(h2pad, srcs, dsts)

    return _k4(a2.reshape(NC, N_ACC, 1), h2p, dis, b2,
               batch.reshape(n, 1), n, g)
